# baseline (speedup 1.0000x reference)
import functools

import jax
import jax.numpy as jnp
from jax import lax
from jax.experimental import pallas as pl
from jax.experimental.pallas import tpu as pltpu
from jax.experimental.pallas import tpu_sc as plsc

NUM_ENTITY = 1000000
DIM = 64
BATCH = 16384

NC = 2
NS = 16
L = 16
NW = NC * NS
B_PER_W = BATCH // NW
IDX_CHUNK = 128
N_CHUNKS = B_PER_W // IDX_CHUNK


def _make_sc_kernel():
    mesh = plsc.VectorSubcoreMesh(core_axis_name="c", subcore_axis_name="s")

    @functools.partial(
        pl.kernel,
        out_type=jax.ShapeDtypeStruct((BATCH,), jnp.float32),
        mesh=mesh,
        compiler_params=pltpu.CompilerParams(
            needs_layout_passes=False, use_tc_tiling_on_sc=False),
        scratch_types=[
            pltpu.VMEM((N_CHUNKS, IDX_CHUNK), jnp.int32),
            pltpu.VMEM((N_CHUNKS, IDX_CHUNK), jnp.int32),
            pltpu.VMEM((B_PER_W, DIM), jnp.float32),
            pltpu.VMEM((B_PER_W, DIM), jnp.float32),
            pltpu.VMEM((B_PER_W,), jnp.float32),
            pltpu.SemaphoreType.DMA,
        ],
    )
    def sc_kernel(idx0_hbm, idx1_hbm, emb_in_hbm, emb_out_hbm, out_hbm,
                  idx0_v, idx1_v, e1_v, e2_v, out_v, sem):
        wid = lax.axis_index("s") * NC + lax.axis_index("c")

        pltpu.sync_copy(idx0_hbm.at[wid], idx0_v)
        pltpu.sync_copy(idx1_hbm.at[wid], idx1_v)

        copies = []
        for j in range(N_CHUNKS):
            dst = pl.ds(j * IDX_CHUNK, IDX_CHUNK)
            copies.append(pltpu.async_copy(
                emb_in_hbm.at[idx0_v.at[j]], e1_v.at[dst], sem))
            copies.append(pltpu.async_copy(
                emb_out_hbm.at[idx1_v.at[j]], e2_v.at[dst], sem))
        for c in copies:
            c.wait()

        lanes = lax.iota(jnp.int32, L)

        def body(g, _):
            base = g * L
            row_vec = jnp.zeros((L,), jnp.float32)
            for k in range(L):
                i = base + k
                acc = jnp.zeros((L,), jnp.float32)
                for c in range(DIM // L):
                    a = e1_v[i, pl.ds(c * L, L)]
                    b = e2_v[i, pl.ds(c * L, L)]
                    diff = a - b
                    acc = acc + diff * diff
                row_vec = jnp.where(lanes == k, -jnp.sum(acc), row_vec)
            out_v[pl.ds(base, L)] = row_vec
            return _

        lax.fori_loop(0, B_PER_W // L, body, 0)

        pltpu.sync_copy(out_v, out_hbm.at[pl.ds(wid * B_PER_W, B_PER_W)])

    return sc_kernel


_sc_kernel = _make_sc_kernel()


@jax.jit
def kernel(idxs, emb_in, emb_out):
    idx0 = idxs[:, 0].reshape(NW, N_CHUNKS, IDX_CHUNK)
    idx1 = idxs[:, 1].reshape(NW, N_CHUNKS, IDX_CHUNK)
    return _sc_kernel(idx0, idx1, emb_in, emb_out)
